# baseline (speedup 1.0000x reference)
import functools

import jax
import jax.numpy as jnp
from jax import lax
from jax.experimental import pallas as pl
from jax.experimental.pallas import tpu as pltpu
from jax.experimental.pallas import tpu_sc as plsc

BATCH = 16384
DIM = 64
HIDDEN = 20
NC = 2
NS = 16
NW = NC * NS
B_PER_W = BATCH // NW
LANES = 16


def _sc_gather_mul(user_hbm, item_hbm, utab_hbm, itab_hbm, x_hbm,
                   uidx, iidx, urows, irows, sem_u, sem_i):
    wid = lax.axis_index("s") * NC + lax.axis_index("c")
    base = wid * B_PER_W
    pltpu.sync_copy(user_hbm.at[pl.ds(base, B_PER_W)], uidx)
    pltpu.sync_copy(item_hbm.at[pl.ds(base, B_PER_W)], iidx)
    cp_u = pltpu.async_copy(utab_hbm.at[uidx], urows, sem_u)
    cp_i = pltpu.async_copy(itab_hbm.at[iidx], irows, sem_i)
    cp_u.wait()
    cp_i.wait()

    def row_body(r, carry):
        for c in range(DIM // LANES):
            sl = pl.ds(c * LANES, LANES)
            urows[r, sl] = urows[r, sl] * irows[r, sl]
        return carry

    lax.fori_loop(0, B_PER_W, row_body, 0)
    pltpu.sync_copy(urows, x_hbm.at[pl.ds(base, B_PER_W)])


@functools.partial(jax.jit, static_argnames=())
def _sc_stage(user, item, user_table, item_table):
    mesh = plsc.VectorSubcoreMesh(core_axis_name="c", subcore_axis_name="s")
    fn = pl.kernel(
        _sc_gather_mul,
        out_type=jax.ShapeDtypeStruct((BATCH, DIM), jnp.float32),
        mesh=mesh,
        scratch_types=[
            pltpu.VMEM((B_PER_W,), jnp.int32),
            pltpu.VMEM((B_PER_W,), jnp.int32),
            pltpu.VMEM((B_PER_W, DIM), jnp.float32),
            pltpu.VMEM((B_PER_W, DIM), jnp.float32),
            pltpu.SemaphoreType.DMA,
            pltpu.SemaphoreType.DMA,
        ],
        compiler_params=pltpu.CompilerParams(use_tc_tiling_on_sc=False),
    )
    return fn(user, item, user_table, item_table)


def _tc_mlp_body(x_ref, w1t_ref, b1_ref, w2t_ref, b2_ref, o_ref):
    x = x_ref[...]
    h = jnp.dot(x, w1t_ref[...], preferred_element_type=jnp.float32)
    h = jnp.maximum(h + b1_ref[...], 0.0)
    z = jnp.dot(h, w2t_ref[...], preferred_element_type=jnp.float32)
    z = z + b2_ref[0, 0]
    o_ref[...] = 1.0 / (1.0 + jnp.exp(-z[:, 0]))


def _tc_mlp(x, W1T, b1r, W2T, b2r):
    blk = 2048
    grid = (BATCH // blk,)
    return pl.pallas_call(
        _tc_mlp_body,
        grid=grid,
        in_specs=[
            pl.BlockSpec((blk, DIM), lambda i: (i, 0)),
            pl.BlockSpec((DIM, HIDDEN), lambda i: (0, 0)),
            pl.BlockSpec((1, HIDDEN), lambda i: (0, 0)),
            pl.BlockSpec((HIDDEN, 1), lambda i: (0, 0)),
            pl.BlockSpec((1, 1), lambda i: (0, 0), memory_space=pltpu.SMEM),
        ],
        out_specs=pl.BlockSpec((blk,), lambda i: (i,)),
        out_shape=jax.ShapeDtypeStruct((BATCH,), jnp.float32),
    )(x, W1T, b1r, W2T, b2r)


def kernel(user, item, user_table, item_table, W1, b1, W2, b2):
    user = user.astype(jnp.int32)
    item = item.astype(jnp.int32)
    x = _sc_stage(user, item, user_table, item_table)
    W1T = W1.T
    b1r = b1.reshape(1, HIDDEN)
    W2T = W2.T
    b2r = b2.reshape(1, 1)
    return _tc_mlp(x, W1T, b1r, W2T, b2r)


# speedup vs baseline: 1.5747x; 1.5747x over previous
import functools

import jax
import jax.numpy as jnp
from jax import lax
from jax.experimental import pallas as pl
from jax.experimental.pallas import tpu as pltpu
from jax.experimental.pallas import tpu_sc as plsc

BATCH = 16384
DIM = 64
HIDDEN = 20
NC = 2
NS = 16
NW = NC * NS
B_PER_W = BATCH // NW
CHUNK = 256
LANES = 16


def _sc_gather_mul(user_hbm, item_hbm, utab_hbm, itab_hbm, x_hbm,
                   uidx, iidx, urows, irows, sem_u, sem_i):
    wid = lax.axis_index("s") * NC + lax.axis_index("c")
    base = wid * B_PER_W
    pltpu.sync_copy(user_hbm.at[pl.ds(base, B_PER_W)], uidx)
    pltpu.sync_copy(item_hbm.at[pl.ds(base, B_PER_W)], iidx)

    for ch in range(B_PER_W // CHUNK):
        off = ch * CHUNK

        def fire(g, carry):
            uv = uidx[pl.ds(off + g * LANES, LANES)]
            iv = iidx[pl.ds(off + g * LANES, LANES)]
            for k in range(LANES):
                j = g * LANES + k
                pltpu.async_copy(utab_hbm.at[pl.ds(uv[k], 1)],
                                 urows.at[pl.ds(j, 1)], sem_u)
                pltpu.async_copy(itab_hbm.at[pl.ds(iv[k], 1)],
                                 irows.at[pl.ds(j, 1)], sem_i)
            return carry

        lax.fori_loop(0, CHUNK // LANES, fire, 0)
        pltpu.make_async_copy(utab_hbm.at[pl.ds(0, CHUNK)], urows, sem_u).wait()
        pltpu.make_async_copy(itab_hbm.at[pl.ds(0, CHUNK)], irows, sem_i).wait()

        def row_body(r, carry):
            for c in range(DIM // LANES):
                sl = pl.ds(c * LANES, LANES)
                urows[r, sl] = urows[r, sl] * irows[r, sl]
            return carry

        lax.fori_loop(0, CHUNK, row_body, 0)
        pltpu.sync_copy(urows, x_hbm.at[pl.ds(base + off, CHUNK)])


@functools.partial(jax.jit, static_argnames=())
def _sc_stage(user, item, user_table, item_table):
    mesh = plsc.VectorSubcoreMesh(core_axis_name="c", subcore_axis_name="s")
    fn = pl.kernel(
        _sc_gather_mul,
        out_type=jax.ShapeDtypeStruct((BATCH, DIM), jnp.float32),
        mesh=mesh,
        scratch_types=[
            pltpu.VMEM((B_PER_W,), jnp.int32),
            pltpu.VMEM((B_PER_W,), jnp.int32),
            pltpu.VMEM((CHUNK, DIM), jnp.float32),
            pltpu.VMEM((CHUNK, DIM), jnp.float32),
            pltpu.SemaphoreType.DMA,
            pltpu.SemaphoreType.DMA,
        ],
    )
    return fn(user, item, user_table, item_table)


def _tc_mlp_body(x_ref, w1t_ref, b1_ref, w2t_ref, b2_ref, o_ref):
    x = x_ref[...]
    h = jnp.dot(x, w1t_ref[...], preferred_element_type=jnp.float32)
    h = jnp.maximum(h + b1_ref[...], 0.0)
    z = jnp.dot(h, w2t_ref[...], preferred_element_type=jnp.float32)
    z = z + b2_ref[0, 0]
    o_ref[...] = 1.0 / (1.0 + jnp.exp(-z[:, 0]))


def _tc_mlp(x, W1T, b1r, W2T, b2r):
    blk = 2048
    grid = (BATCH // blk,)
    return pl.pallas_call(
        _tc_mlp_body,
        grid=grid,
        in_specs=[
            pl.BlockSpec((blk, DIM), lambda i: (i, 0)),
            pl.BlockSpec((DIM, HIDDEN), lambda i: (0, 0)),
            pl.BlockSpec((1, HIDDEN), lambda i: (0, 0)),
            pl.BlockSpec((HIDDEN, 1), lambda i: (0, 0)),
            pl.BlockSpec((1, 1), lambda i: (0, 0), memory_space=pltpu.SMEM),
        ],
        out_specs=pl.BlockSpec((blk,), lambda i: (i,)),
        out_shape=jax.ShapeDtypeStruct((BATCH,), jnp.float32),
    )(x, W1T, b1r, W2T, b2r)


def kernel(user, item, user_table, item_table, W1, b1, W2, b2):
    user = user.astype(jnp.int32)
    item = item.astype(jnp.int32)
    x = _sc_stage(user, item, user_table, item_table)
    W1T = W1.T
    b1r = b1.reshape(1, HIDDEN)
    W2T = W2.T
    b2r = b2.reshape(1, 1)
    return _tc_mlp(x, W1T, b1r, W2T, b2r)
